# baseline (speedup 1.0000x reference)
import functools

import jax
import jax.numpy as jnp
from jax import lax
from jax.experimental import pallas as pl
from jax.experimental.pallas import tpu as pltpu
from jax.experimental.pallas import tpu_sc as plsc

_FIELD_DIM = 100000
_NUM_FIELDS = 26
_BATCH = 16384
_LANES = 16
_NUM_CORES = 2
_NUM_SUBCORES = 16
_NUM_WORKERS = _NUM_CORES * _NUM_SUBCORES
_B_PER_W = _BATCH // _NUM_WORKERS
_VECS = _B_PER_W // _LANES
_GATHER_N = _NUM_FIELDS * _B_PER_W


def _sc_body(xt_hbm, table_hbm, bias_hbm, out_hbm, x_v, idx_v, sidx_v,
             rows_v, acc_sh, sem):
    core = lax.axis_index("c")
    sub = lax.axis_index("s")
    wid = sub * _NUM_CORES + core
    base = wid * _B_PER_W

    pltpu.sync_copy(xt_hbm.at[:, pl.ds(base, _B_PER_W)], x_v)
    pltpu.sync_copy(bias_hbm, acc_sh.at[sub])

    lanes = lax.iota(jnp.int32, _LANES)

    def build(i, _):
        sv = lanes + i * _LANES
        for f in range(_NUM_FIELDS):
            v = x_v[f, pl.ds(i * _LANES, _LANES)]
            idx_v[pl.ds(f * _B_PER_W + i * _LANES, _LANES)] = v + (
                f * _FIELD_DIM)
            sidx_v[pl.ds(f * _B_PER_W + i * _LANES, _LANES)] = sv
        return _

    lax.fori_loop(0, _VECS, build, None)

    pltpu.async_copy(table_hbm.at[idx_v], rows_v, sem).wait()

    pltpu.sync_copy(rows_v, acc_sh.at[sub].at[sidx_v], add=True)

    pltpu.sync_copy(acc_sh.at[sub], out_hbm.at[pl.ds(base, _B_PER_W)])


@functools.partial(jax.jit, static_argnames=())
def kernel(x, W, bias):
    xt = x.T
    table = W.T.reshape(-1)
    bias512 = jnp.broadcast_to(bias, (_B_PER_W,))

    mesh = plsc.VectorSubcoreMesh(core_axis_name="c", subcore_axis_name="s")
    run = pl.kernel(
        _sc_body,
        out_type=jax.ShapeDtypeStruct((_BATCH,), jnp.float32),
        mesh=mesh,
        scratch_types=[
            pltpu.VMEM((_NUM_FIELDS, _B_PER_W), jnp.int32),
            pltpu.VMEM((_GATHER_N,), jnp.int32),
            pltpu.VMEM((_GATHER_N,), jnp.int32),
            pltpu.VMEM((_GATHER_N,), jnp.float32),
            pltpu.VMEM_SHARED((_NUM_SUBCORES, _B_PER_W), jnp.float32),
            pltpu.SemaphoreType.DMA,
        ],
        compiler_params=pltpu.CompilerParams(use_tc_tiling_on_sc=False),
    )
    return run(xt, table, bias512).reshape(_BATCH, 1)
